# baseline (speedup 1.0000x reference)
import jax
import jax.numpy as jnp
from jax import lax
from jax.experimental import pallas as pl
from jax.experimental.pallas import tpu as pltpu, tpu_sc as plsc

_N_VAL = 1000
_ROWS = 4096 * 20
_NW = 32
_RPW = _ROWS // _NW
_RD = 80
_ND = _RPW // _RD
_IC = _RPW // 128


def _sc_body(x_hbm, zeros_hbm, out_hbm, idx_v, buf_v, ones_v, zsem, ssem):
    wid = lax.axis_index("s") * 2 + lax.axis_index("c")
    base_row = wid * _RPW
    out_base = base_row * _N_VAL

    pltpu.sync_copy(zeros_hbm, buf_v)
    pltpu.sync_copy(x_hbm.at[wid], idx_v)

    def fire(i, carry):
        pltpu.async_copy(
            buf_v, out_hbm.at[pl.ds(out_base + i * _RD * _N_VAL, _RD * _N_VAL)],
            zsem,
        )
        return carry
    lax.fori_loop(0, _ND, fire, 0)

    lane = lax.iota(jnp.int32, 16)
    ones16 = jnp.full((16,), 1.0, jnp.float32)
    for k in range(8):
        ones_v[pl.ds(k * 16, 16)] = ones16

    def pos_body(j, carry):
        r = j * 16
        xv = idx_v[r // 128, pl.ds(r % 128, 16)]
        gpos = (base_row + r + lane) * _N_VAL + xv
        idx_v[r // 128, pl.ds(r % 128, 16)] = gpos
        return carry
    lax.fori_loop(0, _RPW // 16, pos_body, 0)

    def drain(i, carry):
        pltpu.make_async_copy(
            buf_v, out_hbm.at[pl.ds(out_base + i * _RD * _N_VAL, _RD * _N_VAL)],
            zsem,
        ).wait()
        return carry
    lax.fori_loop(0, _ND, drain, 0)

    def scat(j, carry):
        pltpu.async_copy(ones_v, out_hbm.at[idx_v.at[j]], ssem)
        return carry
    lax.fori_loop(0, _IC, scat, 0)

    def sdrain(j, carry):
        pltpu.make_async_copy(ones_v, out_hbm.at[idx_v.at[j]], ssem).wait()
        return carry
    lax.fori_loop(0, _IC, sdrain, 0)


def kernel(x):
    x3 = x.reshape(_NW, _RPW // 128, 128)
    zeros = jnp.zeros((_RD * _N_VAL,), jnp.float32)
    mesh = plsc.VectorSubcoreMesh(core_axis_name="c", subcore_axis_name="s")
    out = pl.kernel(
        _sc_body,
        out_type=jax.ShapeDtypeStruct((_ROWS * _N_VAL,), jnp.float32),
        mesh=mesh,
        scratch_types=[
            pltpu.VMEM((_RPW // 128, 128), jnp.int32),
            pltpu.VMEM((_RD * _N_VAL,), jnp.float32),
            pltpu.VMEM((128,), jnp.float32),
            pltpu.SemaphoreType.DMA,
            pltpu.SemaphoreType.DMA,
        ],
        compiler_params=pltpu.CompilerParams(needs_layout_passes=False),
    )(x3, zeros)
    return out.reshape(x.shape + (_N_VAL,))
